# baseline (speedup 1.0000x reference)
import functools

import jax
import jax.numpy as jnp
from jax import lax
from jax.experimental import pallas as pl
from jax.experimental.pallas import tpu as pltpu
from jax.experimental.pallas import tpu_sc as plsc

B = 16384
D = 32
NC = 2
NS = 16
L = 16
NW = NC * NS
CHUNK = B // NW
GSZ = 128
NG = CHUNK // GSZ
PACK = 128 // D

_mesh = plsc.VectorSubcoreMesh(core_axis_name="c", subcore_axis_name="s")


@functools.partial(
    pl.kernel,
    mesh=_mesh,
    out_type=jax.ShapeDtypeStruct((B,), jnp.float32),
    compiler_params=pltpu.CompilerParams(needs_layout_passes=False),
    scratch_types=[
        pltpu.VMEM((NG, GSZ), jnp.int32),
        pltpu.VMEM((NG, GSZ), jnp.int32),
        pltpu.VMEM((NG, GSZ), jnp.int32),
        pltpu.VMEM((NG, GSZ), jnp.int32),
        pltpu.VMEM((GSZ, PACK * D), jnp.float32),
        pltpu.VMEM((GSZ, PACK * D), jnp.float32),
        pltpu.VMEM((CHUNK,), jnp.float32),
        pltpu.SemaphoreType.DMA,
    ],
)
def _gmf(uidx_hbm, iidx_hbm, utab_hbm, itab_hbm, out_hbm,
         uraw_v, iraw_v, uq_v, iq_v, ubuf_v, ibuf_v, out_v, sem):
    wid = lax.axis_index("s") * NC + lax.axis_index("c")
    base = wid * CHUNK

    for k in range(NG):
        pltpu.sync_copy(uidx_hbm.at[pl.ds(base + k * GSZ, GSZ)], uraw_v.at[k])
        pltpu.sync_copy(iidx_hbm.at[pl.ds(base + k * GSZ, GSZ)], iraw_v.at[k])

    def shift(j, carry):
        sl = pl.ds(j * L, L)
        uq_v[carry, sl] = lax.shift_right_logical(uraw_v[carry, sl], 2)
        iq_v[carry, sl] = lax.shift_right_logical(iraw_v[carry, sl], 2)
        return carry

    for k in range(NG):
        lax.fori_loop(0, GSZ // L, shift, k)

    utab = utab_hbm
    itab = itab_hbm

    lanes = lax.iota(jnp.int32, L)

    def make_group(k):
        def group(g, carry):
            cu_vec = (uraw_v[k, pl.ds(g * L, L)] & (PACK - 1)) * D
            ci_vec = (iraw_v[k, pl.ds(g * L, L)] & (PACK - 1)) * D
            acc = jnp.zeros((L,), jnp.float32)
            for l in range(L):
                r = g * L + l
                rvec = jnp.broadcast_to(r.astype(jnp.int32), (L,))
                cu = cu_vec[l] + lanes
                ci = ci_vec[l] + lanes
                u0 = plsc.load_gather(ubuf_v, [rvec, cu])
                u1 = plsc.load_gather(ubuf_v, [rvec, cu + L])
                i0 = plsc.load_gather(ibuf_v, [rvec, ci])
                i1 = plsc.load_gather(ibuf_v, [rvec, ci + L])
                s = jnp.sum(u0 * i0 + u1 * i1)
                acc = jnp.where(lanes == l, s, acc)
            out_v[pl.ds(k * GSZ + g * L, L)] = acc
            return carry
        return group

    for k in range(NG):
        cu = pltpu.async_copy(utab.at[uq_v.at[k]], ubuf_v, sem)
        ci = pltpu.async_copy(itab.at[iq_v.at[k]], ibuf_v, sem)
        cu.wait()
        ci.wait()
        lax.fori_loop(0, GSZ // L, make_group(k), 0)

    pltpu.sync_copy(out_v, out_hbm.at[pl.ds(base, CHUNK)])


def kernel(user_input, item_input, user_table, item_table):
    return _gmf(user_input.astype(jnp.int32), item_input.astype(jnp.int32),
                user_table.reshape(-1, PACK * D),
                item_table.reshape(-1, PACK * D))
